# Initial kernel scaffold; baseline (speedup 1.0000x reference)
import jax
import jax.numpy as jnp
from jax.experimental import pallas as pl


def kernel(seq_word_ids, deno_labels, cono_labels, pretrained_table, deno_table, cono_table, W_deno, b_deno, W_cono, b_cono):
    raise NotImplementedError("write your pallas kernel here")


# speedup vs baseline: 5.2432x; 5.2432x over previous
import functools

import jax
import jax.numpy as jnp
from jax import lax
from jax.experimental import pallas as pl
from jax.experimental.pallas import tpu as pltpu
from jax.experimental.pallas import tpu_sc as plsc

_V = 100000
_D = 128
_B = 4096
_L = 50
_AW = 128
_COS_COL = 88
_NC = 2
_NS = 16
_NW = _NC * _NS
_SEQ_PER_W = _B // _NW
_CHUNK_SEQ = 2
_CHUNK_TOK = _CHUNK_SEQ * _L
_CHUNK_PAD = 104
_NCHUNK = _SEQ_PER_W // _CHUNK_SEQ
_IDS_PER_W = _NCHUNK * _CHUNK_PAD


def _proj_body(d_ref, c_ref, p_ref, w_ref, out_ref):
    d = d_ref[...]
    c = c_ref[...]
    p = p_ref[...]
    w = w_ref[...]
    mm = (jnp.dot(d, w[:_D], preferred_element_type=jnp.float32)
          + jnp.dot(c, w[_D:], preferred_element_type=jnp.float32))
    r = d + c
    num = jnp.sum(r * p, axis=1, keepdims=True)
    den = (jnp.sqrt(jnp.sum(r * r, axis=1, keepdims=True))
           * jnp.sqrt(jnp.sum(p * p, axis=1, keepdims=True)) + 1e-8)
    cos = num / den
    col = lax.broadcasted_iota(jnp.int32, mm.shape, 1)
    out_ref[...] = jnp.where(col == _COS_COL, cos, mm)


def _project(deno_t, cono_t, pre_t, wbig):
    chunk = 2000
    grid = _V // chunk
    return pl.pallas_call(
        _proj_body,
        grid=(grid,),
        in_specs=[
            pl.BlockSpec((chunk, _D), lambda i: (i, 0)),
            pl.BlockSpec((chunk, _D), lambda i: (i, 0)),
            pl.BlockSpec((chunk, _D), lambda i: (i, 0)),
            pl.BlockSpec((2 * _D, _AW), lambda i: (0, 0)),
        ],
        out_specs=pl.BlockSpec((chunk, _AW), lambda i: (i, 0)),
        out_shape=jax.ShapeDtypeStruct((_V, _AW), jnp.float32),
    )(deno_t, cono_t, pre_t, wbig)


def _pool_body(ids_hbm, a_hbm, out_hbm, idx_v, buf_a, buf_b, sv, sem_a, sem_b):
    wid = lax.axis_index("s") * _NC + lax.axis_index("c")
    pltpu.sync_copy(ids_hbm.at[pl.ds(wid * _IDS_PER_W, _IDS_PER_W)], idx_v)

    def gstart(j, buf, sem):
        pltpu.async_copy(a_hbm.at[idx_v.at[pl.ds(j * _CHUNK_PAD, _CHUNK_PAD)]],
                         buf, sem)

    def gwait(buf, sem):
        pltpu.make_async_copy(a_hbm.at[pl.ds(0, _CHUNK_PAD)], buf, sem).wait()

    def accum(j, buf):
        for s in range(_CHUNK_SEQ):
            def tbody(t, acc, s=s):
                row = s * _L + t
                return tuple(acc[r] + buf[row, pl.ds(r * 16, 16)]
                             for r in range(_AW // 16))
            acc0 = tuple(jnp.zeros((16,), jnp.float32)
                         for _ in range(_AW // 16))
            acc = lax.fori_loop(0, _L, tbody, acc0)
            sl = j * _CHUNK_SEQ + s
            for r in range(_AW // 16):
                sv[sl, pl.ds(r * 16, 16)] = acc[r]

    gstart(0, buf_a, sem_a)

    def body(jj, carry):
        j0 = jj * 2
        gstart(j0 + 1, buf_b, sem_b)
        gwait(buf_a, sem_a)
        accum(j0, buf_a)

        @pl.when(jj < _NCHUNK // 2 - 1)
        def _():
            gstart(j0 + 2, buf_a, sem_a)

        gwait(buf_b, sem_b)
        accum(j0 + 1, buf_b)
        return carry

    lax.fori_loop(0, _NCHUNK // 2, body, 0)
    pltpu.sync_copy(sv, out_hbm.at[pl.ds(wid * _SEQ_PER_W, _SEQ_PER_W)])


_pool = functools.partial(
    pl.kernel,
    out_type=jax.ShapeDtypeStruct((_B, _AW), jnp.float32),
    mesh=plsc.VectorSubcoreMesh(core_axis_name="c", subcore_axis_name="s"),
    scratch_types=[
        pltpu.VMEM((_IDS_PER_W,), jnp.int32),
        pltpu.VMEM((_CHUNK_PAD, _AW), jnp.float32),
        pltpu.VMEM((_CHUNK_PAD, _AW), jnp.float32),
        pltpu.VMEM((_SEQ_PER_W, _AW), jnp.float32),
        pltpu.SemaphoreType.DMA,
        pltpu.SemaphoreType.DMA,
    ],
)(_pool_body)


def _loss_body(s_ref, dlab_ref, clab_ref, bias_ref, out_ref):
    S = s_ref[...]
    m = S * (1.0 / _L) + bias_ref[...]
    col = lax.broadcasted_iota(jnp.int32, m.shape, 1)
    dl = dlab_ref[...]
    cl = clab_ref[...]

    def lse_group(s0, w):
        mask = (col >= s0) & (col < s0 + w)
        mx = jnp.max(jnp.where(mask, m, -jnp.inf), axis=1, keepdims=True)
        e = jnp.where(mask, jnp.exp(m - mx), 0.0)
        return mx + jnp.log(jnp.sum(e, axis=1, keepdims=True)), mask

    lse_d0, _ = lse_group(0, 41)
    lse_c0, mask_c0 = lse_group(41, 2)
    lse_d1, mask_d1 = lse_group(43, 41)
    lse_c1, _ = lse_group(84, 2)
    Bf = float(_B)

    def pick(lse, labcol):
        return jnp.sum(jnp.where(col == labcol, m - lse, 0.0))

    DS_dp = -pick(lse_d0, dl) / Bf
    DS_cp = -pick(lse_c0, cl + 41) / Bf
    CS_cp = -pick(lse_c1, cl + 84) / Bf
    sum_clp0 = jnp.sum(jnp.where(mask_c0, m - lse_c0, 0.0))
    sum_dlp1 = jnp.sum(jnp.where(mask_d1, m - lse_d1, 0.0))
    u2, u41 = 0.5, 1.0 / 41.0
    DS_ca = 2 * u2 * jnp.log(u2) - u2 * sum_clp0 / Bf
    CS_da = 41 * u41 * jnp.log(u41) - u41 * sum_dlp1 / Bf

    def sig(x):
        return 1.0 / (1.0 + jnp.exp(-x))

    L_DS = sig(DS_dp) + sig(DS_ca)
    L_CS = sig(CS_da) + sig(CS_cp)
    cos_total = jnp.sum(jnp.where(col == _COS_COL, S, 0.0))
    L_R = 1.0 - cos_total / (Bf * _L)
    L_joint = L_DS + L_CS + L_R

    vals = (L_joint, L_DS, DS_dp, DS_cp, DS_ca, L_CS, CS_da, CS_cp, L_R)
    orow = lax.broadcasted_iota(jnp.int32, (8, 128), 0)
    ocol = lax.broadcasted_iota(jnp.int32, (8, 128), 1)
    out = jnp.zeros((8, 128), jnp.float32)
    for k, v in enumerate(vals):
        out = jnp.where((orow == 0) & (ocol == k), v, out)
    out_ref[...] = out


def _losses(S, dlab2, clab2, bias):
    return pl.pallas_call(
        _loss_body,
        out_shape=jax.ShapeDtypeStruct((8, 128), jnp.float32),
    )(S, dlab2, clab2, bias)


def kernel(seq_word_ids, deno_labels, cono_labels, pretrained_table,
           deno_table, cono_table, W_deno, b_deno, W_cono, b_cono):
    f32 = jnp.float32
    wbig = jnp.concatenate([
        jnp.concatenate([W_deno, W_cono, jnp.zeros((_D, _AW - 43), f32)], axis=1),
        jnp.concatenate([jnp.zeros((_D, 43), f32), W_deno, W_cono,
                         jnp.zeros((_D, _AW - 86), f32)], axis=1),
    ], axis=0)

    A = _project(deno_table, cono_table, pretrained_table, wbig)

    ids_padded = jnp.pad(
        seq_word_ids.astype(jnp.int32).reshape(-1, _CHUNK_TOK),
        ((0, 0), (0, _CHUNK_PAD - _CHUNK_TOK)),
    ).reshape(-1)
    S = _pool(ids_padded, A)

    bias = jnp.concatenate([b_deno, b_cono, b_deno, b_cono,
                            jnp.zeros((_AW - 86,), f32)]).reshape(1, _AW)
    out = _losses(S, deno_labels.astype(jnp.int32).reshape(_B, 1),
                  cono_labels.astype(jnp.int32).reshape(_B, 1), bias)
    return out[0, :9]
